# Initial kernel scaffold; baseline (speedup 1.0000x reference)
import functools

import jax
import jax.numpy as jnp
import jax.scipy.linalg as jsl
from jax import lax
from jax.experimental import pallas as pl
from jax.experimental.pallas import tpu as pltpu
from jax.experimental.pallas import tpu_sc as plsc

_F = 50000
_L = 5000
_P = 8
_K = 80
_D = 32
_ITER = 8
_NW = 32


def _sc_gather(table, idx, fire=4):
    R = idx.shape[0]
    Dd = table.shape[1]
    assert R % 8 == 0
    span = -(-R // _NW)
    span = -(-span // 8) * 8
    min_count = R - (_NW - 1) * span
    assert min_count > 0
    ch = min(128, (min_count // 8) * 8)
    idx_pad = jnp.pad(idx, (0, _NW * span - R))
    mesh = plsc.VectorSubcoreMesh(core_axis_name="c", subcore_axis_name="s")

    @functools.partial(
        pl.kernel,
        mesh=mesh,
        out_type=jax.ShapeDtypeStruct((R, Dd), jnp.float32),
        scratch_types=(
            [pltpu.VMEM((span,), jnp.int32)]
            + [pltpu.VMEM((ch, Dd), jnp.float32) for _ in range(fire)]
            + [pltpu.SemaphoreType.DMA]
        ),
    )
    def gather_kernel(table_hbm, idx_hbm, out_hbm, idx_v, *rest):
        bufs, sem = rest[:fire], rest[fire]
        wid = lax.axis_index("s") * 2 + lax.axis_index("c")
        base = wid * span
        count = jnp.minimum(base + span, R) - base
        pltpu.sync_copy(idx_hbm.at[pl.ds(base, span)], idx_v)
        nfull = count // ch
        rem = count - nfull * ch
        nf = nfull // fire

        def gather_chunk(pos, buf):
            return pltpu.async_copy(
                table_hbm.at[idx_v.at[pl.ds(pos, ch)]], buf, sem)

        def store_chunk(pos, buf):
            pltpu.sync_copy(buf, out_hbm.at[pl.ds(base + pos, ch)])

        def body(j, carry):
            p0 = j * (fire * ch)
            descs = [gather_chunk(p0 + u * ch, bufs[u]) for u in range(fire)]
            for d in descs:
                d.wait()
            for u in range(fire):
                store_chunk(p0 + u * ch, bufs[u])
            return carry

        lax.fori_loop(0, nf, body, 0)
        done = nf * fire
        for u in range(fire - 1):
            @pl.when(done + u < nfull)
            def _(u=u):
                p = (done + u) * ch
                gather_chunk(p, bufs[0]).wait()
                store_chunk(p, bufs[0])

        @pl.when(rem > 0)
        def _():
            p = count - ch
            gather_chunk(p, bufs[0]).wait()
            store_chunk(p, bufs[0])

    return gather_kernel(table, idx_pad)


def _full(shape):
    return pl.BlockSpec(shape, lambda i: tuple(0 for _ in shape))


def _mlp_embed(x, W1, b1, W2, b2, bf):
    n, din = x.shape
    d1, d2 = W1.shape[1], W2.shape[1]

    def body(x_ref, w1, bb1, w2, bb2, o_ref):
        h = jnp.maximum(
            jnp.dot(x_ref[...], w1[...],
                    preferred_element_type=jnp.float32) + bb1[...], 0.0)
        o_ref[...] = jnp.maximum(
            jnp.dot(h, w2[...],
                    preferred_element_type=jnp.float32) + bb2[...], 0.0)

    return pl.pallas_call(
        body,
        grid=(n // bf,),
        in_specs=[pl.BlockSpec((bf, din), lambda i: (i, 0)),
                  _full((din, d1)), _full((1, d1)),
                  _full((d1, d2)), _full((1, d2))],
        out_specs=pl.BlockSpec((bf, d2), lambda i: (i, 0)),
        out_shape=jax.ShapeDtypeStruct((n, d2), jnp.float32),
    )(x, W1, b1.reshape(1, -1), W2, b2.reshape(1, -1))


def _init_small(trg, cap, boh, pr):
    def body(trg_ref, cap_ref, boh_ref, lw1, lb1, lw2, lb2,
             qw1, qb1, qw2, qb2, wpq, wpl, bp, link_ref, queue_ref, t_ref):
        load = jnp.sum(trg_ref[...], axis=1, keepdims=True) / cap_ref[...]
        h = jnp.maximum(jnp.dot(load, lw1[...],
                                preferred_element_type=jnp.float32) + lb1[...], 0.0)
        ls = jnp.maximum(jnp.dot(h, lw2[...],
                                 preferred_element_type=jnp.float32) + lb2[...], 0.0)
        h = jnp.maximum(jnp.dot(boh_ref[...], qw1[...],
                                preferred_element_type=jnp.float32) + qb1[...], 0.0)
        qs = jnp.maximum(jnp.dot(h, qw2[...],
                                 preferred_element_type=jnp.float32) + qb2[...], 0.0)
        link_ref[...] = ls
        queue_ref[...] = qs
        t_ref[...] = (jnp.dot(qs, wpq[...], preferred_element_type=jnp.float32)
                      + jnp.dot(ls, wpl[...], preferred_element_type=jnp.float32)
                      + bp[...])

    return pl.pallas_call(
        body,
        grid=(1,),
        in_specs=[_full((_L, _K)), _full((_L, 1)), _full((_L, 2)),
                  _full((1, _D)), _full((1, _D)), _full((_D, _D)), _full((1, _D)),
                  _full((2, _D)), _full((1, _D)), _full((_D, _D)), _full((1, _D)),
                  _full((_D, 3 * _D)), _full((_D, 3 * _D)), _full((1, 3 * _D))],
        out_specs=(_full((_L, _D)), _full((_L, _D)), _full((_L, 3 * _D))),
        out_shape=(jax.ShapeDtypeStruct((_L, _D), jnp.float32),
                   jax.ShapeDtypeStruct((_L, _D), jnp.float32),
                   jax.ShapeDtypeStruct((_L, 3 * _D), jnp.float32)),
    )(trg, cap, boh,
      pr['le_W1'], pr['le_b1'].reshape(1, -1), pr['le_W2'], pr['le_b2'].reshape(1, -1),
      pr['qe_W1'], pr['qe_b1'].reshape(1, -1), pr['qe_W2'], pr['qe_b2'].reshape(1, -1),
      pr['gru_p_W'][:_D], pr['gru_p_W'][_D:], pr['gru_p_b'].reshape(1, -1))


def _gru_gates(xg, h, hg):
    z = jax.nn.sigmoid(xg[:, 0:_D] + hg[:, 0:_D])
    r = jax.nn.sigmoid(xg[:, _D:2 * _D] + hg[:, _D:2 * _D])
    c = jnp.tanh(xg[:, 2 * _D:3 * _D] + r * hg[:, 2 * _D:3 * _D])
    return z * h + (1.0 - z) * c


def _path_gru(xg, h0, U, bf=2500):
    def body(xg_ref, h_ref, u_ref, pss_ref, ho_ref):
        h = h_ref[...]
        u = u_ref[...]
        pss_ref[:, 0:_D] = h
        for t in range(_P):
            xgt = xg_ref[:, t * 96:(t + 1) * 96]
            hg = jnp.dot(h, u, preferred_element_type=jnp.float32)
            h = _gru_gates(xgt, h, hg)
            pss_ref[:, (t + 1) * _D:(t + 2) * _D] = h
        ho_ref[...] = h

    return pl.pallas_call(
        body,
        grid=(_F // bf,),
        in_specs=[pl.BlockSpec((bf, _P * 96), lambda i: (i, 0)),
                  pl.BlockSpec((bf, _D), lambda i: (i, 0)),
                  _full((_D, 3 * _D))],
        out_specs=(pl.BlockSpec((bf, 9 * _D), lambda i: (i, 0)),
                   pl.BlockSpec((bf, _D), lambda i: (i, 0))),
        out_shape=(jax.ShapeDtypeStruct((_F, 9 * _D), jnp.float32),
                   jax.ShapeDtypeStruct((_F, _D), jnp.float32)),
    )(xg, h0, U)


def _queue_gru(psr, q, S, Wq, Uq, bq, Wl, bl, bl_blk=1000):
    def body(psr_ref, q_ref, s_ref, wq, uq, bbq, wl, bbl, qn_ref, qw2_ref):
        psum = jnp.dot(psr_ref[...], s_ref[...],
                       preferred_element_type=jnp.float32)
        xg = jnp.dot(psum, wq[...], preferred_element_type=jnp.float32) + bbq[...]
        qh = q_ref[...]
        hg = jnp.dot(qh, uq[...], preferred_element_type=jnp.float32)
        qn = _gru_gates(xg, qh, hg)
        qn_ref[...] = qn
        qw2_ref[...] = jnp.dot(qn, wl[...],
                               preferred_element_type=jnp.float32) + bbl[...]

    return pl.pallas_call(
        body,
        grid=(_L // bl_blk,),
        in_specs=[pl.BlockSpec((bl_blk, _K * _D), lambda i: (i, 0)),
                  pl.BlockSpec((bl_blk, _D), lambda i: (i, 0)),
                  _full((_K * _D, _D)),
                  _full((_D, 3 * _D)), _full((_D, 3 * _D)), _full((1, 3 * _D)),
                  _full((_D, 3 * _D)), _full((1, 3 * _D))],
        out_specs=(pl.BlockSpec((bl_blk, _D), lambda i: (i, 0)),
                   pl.BlockSpec((bl_blk, 3 * _D), lambda i: (i, 0))),
        out_shape=(jax.ShapeDtypeStruct((_L, _D), jnp.float32),
                   jax.ShapeDtypeStruct((_L, 3 * _D), jnp.float32)),
    )(psr, q, S, Wq, Uq, bq.reshape(1, -1), Wl, bl.reshape(1, -1))


def _link_gru(qg2, lnk, qn, Ul, WpQ, WpL, bp):
    def body(qg_ref, l_ref, qn_ref, ul, wpq, wpl, bbp, lo_ref, t_ref):
        hl = l_ref[...]
        hg = jnp.dot(hl, ul[...], preferred_element_type=jnp.float32)
        ln = _gru_gates(qg_ref[...], hl, hg)
        lo_ref[...] = ln
        t_ref[...] = (jnp.dot(qn_ref[...], wpq[...],
                              preferred_element_type=jnp.float32)
                      + jnp.dot(ln, wpl[...], preferred_element_type=jnp.float32)
                      + bbp[...])

    return pl.pallas_call(
        body,
        grid=(1,),
        in_specs=[_full((_L, 3 * _D)), _full((_L, _D)), _full((_L, _D)),
                  _full((_D, 3 * _D)), _full((_D, 3 * _D)), _full((_D, 3 * _D)),
                  _full((1, 3 * _D))],
        out_specs=(_full((_L, _D)), _full((_L, 3 * _D))),
        out_shape=(jax.ShapeDtypeStruct((_L, _D), jnp.float32),
                   jax.ShapeDtypeStruct((_L, 3 * _D), jnp.float32)),
    )(qg2, lnk, qn, Ul, WpQ, WpL, bp.reshape(1, -1))


def _readout(pss, capg, tr, pk, W1b, b1b, W2b, b2b, W3b, b3b, bf=2500):
    def body(pss_ref, cg_ref, tr_ref, pk_ref, w1, bb1, w2, bb2, w3, bb3, o_ref):
        x = pss_ref[:, _D:9 * _D]
        h1 = jnp.maximum(jnp.dot(x, w1[...],
                                 preferred_element_type=jnp.float32) + bb1[...], 0.0)
        h2 = jnp.maximum(jnp.dot(h1, w2[...],
                                 preferred_element_type=jnp.float32) + bb2[...], 0.0)
        occ = jnp.dot(h2, w3[...], preferred_element_type=jnp.float32) + bb3[...]
        cg = cg_ref[...] * 1e9
        qd = jnp.sum(occ / cg, axis=1, keepdims=True)
        inv = jnp.sum(1.0 / cg, axis=1, keepdims=True)
        o_ref[...] = qd + (tr_ref[...] / pk_ref[...]) * inv

    return pl.pallas_call(
        body,
        grid=(_F // bf,),
        in_specs=[pl.BlockSpec((bf, 9 * _D), lambda i: (i, 0)),
                  pl.BlockSpec((bf, _P), lambda i: (i, 0)),
                  pl.BlockSpec((bf, 1), lambda i: (i, 0)),
                  pl.BlockSpec((bf, 1), lambda i: (i, 0)),
                  _full((8 * _D, 128)), _full((1, 128)),
                  _full((128, 128)), _full((1, 128)),
                  _full((128, _P)), _full((1, _P))],
        out_specs=pl.BlockSpec((bf, 1), lambda i: (i, 0)),
        out_shape=jax.ShapeDtypeStruct((_F, 1), jnp.float32),
    )(pss, capg, tr, pk, W1b, b1b, W2b, b2b, W3b, b3b)


def kernel(flow_traffic, flow_packets, flow_time_dist, flow_lambda,
           flow_ON_bits_rate, flow_ON_time, flow_OFF_time, link_capacity,
           buffer_type, link_to_path, path_to_link, queue_to_link, params):
    pr = params
    f32 = jnp.float32

    dist_oh = jax.nn.one_hot(flow_time_dist[:, 0], 3, dtype=f32)
    path_in = jnp.concatenate(
        [flow_traffic, flow_packets, dist_oh, flow_lambda,
         flow_ON_bits_rate, flow_ON_time, flow_OFF_time], axis=1)
    boh = jax.nn.one_hot(buffer_type[:, 0], 2, dtype=f32)

    i_l2p = link_to_path.reshape(-1).astype(jnp.int32)
    p2l_f = path_to_link[:, :, 0].astype(jnp.int32)
    p2l_pos = path_to_link[:, :, 1].astype(jnp.int32)
    i_f9 = (p2l_f * 9 + p2l_pos).reshape(-1)
    i_trf = p2l_f.reshape(-1)
    i_q2l = queue_to_link[:, 0].astype(jnp.int32)

    tr16 = jnp.pad(flow_traffic, ((0, 0), (0, 15)))
    cap16 = jnp.pad(link_capacity, ((0, 0), (0, 15)))
    trg = _sc_gather(tr16, i_trf)[:, :1].reshape(_L, _K)
    capg = _sc_gather(cap16, i_l2p)[:, :1].reshape(_F, _P)

    path_state = _mlp_embed(path_in, pr['pe_W1'], pr['pe_b1'],
                            pr['pe_W2'], pr['pe_b2'], bf=2500)
    link_state, queue_state, T = _init_small(trg, link_capacity, boh, pr)

    S = jnp.tile(jnp.eye(_D, dtype=f32), (_K, 1))
    pss = None
    for _ in range(_ITER):
        xg = _sc_gather(T, i_l2p).reshape(_F, _P * 96)
        pss, path_state = _path_gru(xg, path_state, pr['gru_p_U'])
        psr = _sc_gather(pss.reshape(_F * 9, _D), i_f9).reshape(_L, _K * _D)
        queue_state, qw2 = _queue_gru(psr, queue_state, S, pr['gru_q_W'],
                                      pr['gru_q_U'], pr['gru_q_b'],
                                      pr['gru_l_W'], pr['gru_l_b'])
        qg2 = _sc_gather(qw2, i_q2l)
        link_state, T = _link_gru(qg2, link_state, queue_state, pr['gru_l_U'],
                                  pr['gru_p_W'][:_D], pr['gru_p_W'][_D:],
                                  pr['gru_p_b'])

    W1b = jsl.block_diag(*([pr['ro_W1']] * _P))
    b1b = jnp.tile(pr['ro_b1'], _P).reshape(1, -1)
    W2b = jsl.block_diag(*([pr['ro_W2']] * _P))
    b2b = jnp.tile(pr['ro_b2'], _P).reshape(1, -1)
    W3b = jsl.block_diag(*([pr['ro_W3']] * _P))
    b3b = jnp.tile(pr['ro_b3'], _P).reshape(1, -1)
    return _readout(pss, capg, flow_traffic, flow_packets,
                    W1b, b1b, W2b, b2b, W3b, b3b)


# speedup vs baseline: 4.5086x; 4.5086x over previous
import functools

import jax
import jax.numpy as jnp
import jax.scipy.linalg as jsl
from jax import lax
from jax.experimental import pallas as pl
from jax.experimental.pallas import tpu as pltpu
from jax.experimental.pallas import tpu_sc as plsc

_F = 50000
_L = 5000
_P = 8
_K = 80
_D = 32
_ITER = 8
_NW = 32


def _sc_gather(table, idx, fire=4):
    R = idx.shape[0]
    Dd = table.shape[1]
    assert R % 8 == 0
    span = -(-R // _NW)
    span = -(-span // 8) * 8
    min_count = R - (_NW - 1) * span
    assert min_count > 0
    ch = min(128, (min_count // 8) * 8)
    idx_pad = jnp.pad(idx, (0, _NW * span - R))
    mesh = plsc.VectorSubcoreMesh(core_axis_name="c", subcore_axis_name="s")

    @functools.partial(
        pl.kernel,
        mesh=mesh,
        out_type=jax.ShapeDtypeStruct((R, Dd), jnp.float32),
        compiler_params=pltpu.CompilerParams(use_tc_tiling_on_sc=False),
        scratch_types=(
            [pltpu.VMEM((span,), jnp.int32)]
            + [pltpu.VMEM((ch, Dd), jnp.float32) for _ in range(fire)]
            + [pltpu.SemaphoreType.DMA]
        ),
    )
    def gather_kernel(table_hbm, idx_hbm, out_hbm, idx_v, *rest):
        bufs, sem = rest[:fire], rest[fire]
        wid = lax.axis_index("s") * 2 + lax.axis_index("c")
        base = wid * span
        count = jnp.minimum(base + span, R) - base
        pltpu.sync_copy(idx_hbm.at[pl.ds(base, span)], idx_v)
        nfull = count // ch
        rem = count - nfull * ch
        nf = nfull // fire

        def gather_chunk(pos, buf):
            return pltpu.async_copy(
                table_hbm.at[idx_v.at[pl.ds(pos, ch)]], buf, sem)

        def store_chunk(pos, buf):
            pltpu.sync_copy(buf, out_hbm.at[pl.ds(base + pos, ch)])

        def body(j, carry):
            p0 = j * (fire * ch)
            descs = [gather_chunk(p0 + u * ch, bufs[u]) for u in range(fire)]
            for d in descs:
                d.wait()
            for u in range(fire):
                store_chunk(p0 + u * ch, bufs[u])
            return carry

        lax.fori_loop(0, nf, body, 0)
        done = nf * fire
        for u in range(fire - 1):
            @pl.when(done + u < nfull)
            def _(u=u):
                p = (done + u) * ch
                gather_chunk(p, bufs[0]).wait()
                store_chunk(p, bufs[0])

        @pl.when(rem > 0)
        def _():
            p = count - ch
            gather_chunk(p, bufs[0]).wait()
            store_chunk(p, bufs[0])

    return gather_kernel(table, idx_pad)


def _full(shape):
    return pl.BlockSpec(shape, lambda i: tuple(0 for _ in shape))


def _mlp_embed(x, W1, b1, W2, b2, bf):
    n, din = x.shape
    d1, d2 = W1.shape[1], W2.shape[1]

    def body(x_ref, w1, bb1, w2, bb2, o_ref):
        h = jnp.maximum(
            jnp.dot(x_ref[...], w1[...],
                    preferred_element_type=jnp.float32) + bb1[...], 0.0)
        o_ref[...] = jnp.maximum(
            jnp.dot(h, w2[...],
                    preferred_element_type=jnp.float32) + bb2[...], 0.0)

    return pl.pallas_call(
        body,
        grid=(n // bf,),
        in_specs=[pl.BlockSpec((bf, din), lambda i: (i, 0)),
                  _full((din, d1)), _full((1, d1)),
                  _full((d1, d2)), _full((1, d2))],
        out_specs=pl.BlockSpec((bf, d2), lambda i: (i, 0)),
        out_shape=jax.ShapeDtypeStruct((n, d2), jnp.float32),
    )(x, W1, b1.reshape(1, -1), W2, b2.reshape(1, -1))


def _init_small(trg, cap, boh, pr):
    def body(trg_ref, cap_ref, boh_ref, lw1, lb1, lw2, lb2,
             qw1, qb1, qw2, qb2, wpq, wpl, bp, link_ref, queue_ref, t_ref):
        load = jnp.sum(trg_ref[...], axis=1, keepdims=True) / cap_ref[...]
        h = jnp.maximum(jnp.dot(load, lw1[...],
                                preferred_element_type=jnp.float32) + lb1[...], 0.0)
        ls = jnp.maximum(jnp.dot(h, lw2[...],
                                 preferred_element_type=jnp.float32) + lb2[...], 0.0)
        h = jnp.maximum(jnp.dot(boh_ref[...], qw1[...],
                                preferred_element_type=jnp.float32) + qb1[...], 0.0)
        qs = jnp.maximum(jnp.dot(h, qw2[...],
                                 preferred_element_type=jnp.float32) + qb2[...], 0.0)
        link_ref[...] = ls
        queue_ref[...] = qs
        t_ref[...] = (jnp.dot(qs, wpq[...], preferred_element_type=jnp.float32)
                      + jnp.dot(ls, wpl[...], preferred_element_type=jnp.float32)
                      + bp[...])

    return pl.pallas_call(
        body,
        grid=(1,),
        in_specs=[_full((_L, _K)), _full((_L, 1)), _full((_L, 2)),
                  _full((1, _D)), _full((1, _D)), _full((_D, _D)), _full((1, _D)),
                  _full((2, _D)), _full((1, _D)), _full((_D, _D)), _full((1, _D)),
                  _full((_D, 3 * _D)), _full((_D, 3 * _D)), _full((1, 3 * _D))],
        out_specs=(_full((_L, _D)), _full((_L, _D)), _full((_L, 3 * _D))),
        out_shape=(jax.ShapeDtypeStruct((_L, _D), jnp.float32),
                   jax.ShapeDtypeStruct((_L, _D), jnp.float32),
                   jax.ShapeDtypeStruct((_L, 3 * _D), jnp.float32)),
    )(trg, cap, boh,
      pr['le_W1'], pr['le_b1'].reshape(1, -1), pr['le_W2'], pr['le_b2'].reshape(1, -1),
      pr['qe_W1'], pr['qe_b1'].reshape(1, -1), pr['qe_W2'], pr['qe_b2'].reshape(1, -1),
      pr['gru_p_W'][:_D], pr['gru_p_W'][_D:], pr['gru_p_b'].reshape(1, -1))


def _gru_gates(xg, h, hg):
    z = jax.nn.sigmoid(xg[:, 0:_D] + hg[:, 0:_D])
    r = jax.nn.sigmoid(xg[:, _D:2 * _D] + hg[:, _D:2 * _D])
    c = jnp.tanh(xg[:, 2 * _D:3 * _D] + r * hg[:, 2 * _D:3 * _D])
    return z * h + (1.0 - z) * c


def _path_gru(xg, h0, U, bf=2000):
    def body(xg_ref, h_ref, u_ref, pss_ref, ho_ref):
        h = h_ref[...]
        u = u_ref[...]
        pss_ref[:, 0:_D] = h
        for t in range(_P):
            xgt = xg_ref[:, t * 96:(t + 1) * 96]
            hg = jnp.dot(h, u, preferred_element_type=jnp.float32)
            h = _gru_gates(xgt, h, hg)
            pss_ref[:, (t + 1) * _D:(t + 2) * _D] = h
        ho_ref[...] = h

    return pl.pallas_call(
        body,
        grid=(_F // bf,),
        in_specs=[pl.BlockSpec((bf, _P * 96), lambda i: (i, 0)),
                  pl.BlockSpec((bf, _D), lambda i: (i, 0)),
                  _full((_D, 3 * _D))],
        out_specs=(pl.BlockSpec((bf, 9 * _D), lambda i: (i, 0)),
                   pl.BlockSpec((bf, _D), lambda i: (i, 0))),
        out_shape=(jax.ShapeDtypeStruct((_F, 9 * _D), jnp.float32),
                   jax.ShapeDtypeStruct((_F, _D), jnp.float32)),
    )(xg, h0, U)


def _queue_gru(psr, q, S, Wq, Uq, bq, Wl, bl, bl_blk=1000):
    def body(psr_ref, q_ref, s_ref, wq, uq, bbq, wl, bbl, qn_ref, qw2_ref):
        psum = jnp.dot(psr_ref[...], s_ref[...],
                       preferred_element_type=jnp.float32)
        xg = jnp.dot(psum, wq[...], preferred_element_type=jnp.float32) + bbq[...]
        qh = q_ref[...]
        hg = jnp.dot(qh, uq[...], preferred_element_type=jnp.float32)
        qn = _gru_gates(xg, qh, hg)
        qn_ref[...] = qn
        qw2_ref[...] = jnp.dot(qn, wl[...],
                               preferred_element_type=jnp.float32) + bbl[...]

    return pl.pallas_call(
        body,
        grid=(_L // bl_blk,),
        in_specs=[pl.BlockSpec((bl_blk, _K * _D), lambda i: (i, 0)),
                  pl.BlockSpec((bl_blk, _D), lambda i: (i, 0)),
                  _full((_K * _D, _D)),
                  _full((_D, 3 * _D)), _full((_D, 3 * _D)), _full((1, 3 * _D)),
                  _full((_D, 3 * _D)), _full((1, 3 * _D))],
        out_specs=(pl.BlockSpec((bl_blk, _D), lambda i: (i, 0)),
                   pl.BlockSpec((bl_blk, 3 * _D), lambda i: (i, 0))),
        out_shape=(jax.ShapeDtypeStruct((_L, _D), jnp.float32),
                   jax.ShapeDtypeStruct((_L, 3 * _D), jnp.float32)),
    )(psr, q, S, Wq, Uq, bq.reshape(1, -1), Wl, bl.reshape(1, -1))


def _link_gru(qg2, lnk, qn, Ul, WpQ, WpL, bp):
    def body(qg_ref, l_ref, qn_ref, ul, wpq, wpl, bbp, lo_ref, t_ref):
        hl = l_ref[...]
        hg = jnp.dot(hl, ul[...], preferred_element_type=jnp.float32)
        ln = _gru_gates(qg_ref[...], hl, hg)
        lo_ref[...] = ln
        t_ref[...] = (jnp.dot(qn_ref[...], wpq[...],
                              preferred_element_type=jnp.float32)
                      + jnp.dot(ln, wpl[...], preferred_element_type=jnp.float32)
                      + bbp[...])

    return pl.pallas_call(
        body,
        grid=(1,),
        in_specs=[_full((_L, 3 * _D)), _full((_L, _D)), _full((_L, _D)),
                  _full((_D, 3 * _D)), _full((_D, 3 * _D)), _full((_D, 3 * _D)),
                  _full((1, 3 * _D))],
        out_specs=(_full((_L, _D)), _full((_L, 3 * _D))),
        out_shape=(jax.ShapeDtypeStruct((_L, _D), jnp.float32),
                   jax.ShapeDtypeStruct((_L, 3 * _D), jnp.float32)),
    )(qg2, lnk, qn, Ul, WpQ, WpL, bp.reshape(1, -1))


def _readout(pss, capg, tr, pk, W1b, b1b, W2b, b2b, W3b, b3b, bf=2000):
    def body(pss_ref, cg_ref, tr_ref, pk_ref, w1, bb1, w2, bb2, w3, bb3, o_ref):
        x = pss_ref[:, _D:9 * _D]
        h1 = jnp.maximum(jnp.dot(x, w1[...],
                                 preferred_element_type=jnp.float32) + bb1[...], 0.0)
        h2 = jnp.maximum(jnp.dot(h1, w2[...],
                                 preferred_element_type=jnp.float32) + bb2[...], 0.0)
        occ = jnp.dot(h2, w3[...], preferred_element_type=jnp.float32) + bb3[...]
        cg = cg_ref[...] * 1e9
        qd = jnp.sum(occ / cg, axis=1, keepdims=True)
        inv = jnp.sum(1.0 / cg, axis=1, keepdims=True)
        o_ref[...] = qd + (tr_ref[...] / pk_ref[...]) * inv

    return pl.pallas_call(
        body,
        grid=(_F // bf,),
        in_specs=[pl.BlockSpec((bf, 9 * _D), lambda i: (i, 0)),
                  pl.BlockSpec((bf, _P), lambda i: (i, 0)),
                  pl.BlockSpec((bf, 1), lambda i: (i, 0)),
                  pl.BlockSpec((bf, 1), lambda i: (i, 0)),
                  _full((8 * _D, 128)), _full((1, 128)),
                  _full((128, 128)), _full((1, 128)),
                  _full((128, _P)), _full((1, _P))],
        out_specs=pl.BlockSpec((bf, 1), lambda i: (i, 0)),
        out_shape=jax.ShapeDtypeStruct((_F, 1), jnp.float32),
    )(pss, capg, tr, pk, W1b, b1b, W2b, b2b, W3b, b3b)


def kernel(flow_traffic, flow_packets, flow_time_dist, flow_lambda,
           flow_ON_bits_rate, flow_ON_time, flow_OFF_time, link_capacity,
           buffer_type, link_to_path, path_to_link, queue_to_link, params):
    pr = params
    f32 = jnp.float32

    dist_oh = jax.nn.one_hot(flow_time_dist[:, 0], 3, dtype=f32)
    path_in = jnp.concatenate(
        [flow_traffic, flow_packets, dist_oh, flow_lambda,
         flow_ON_bits_rate, flow_ON_time, flow_OFF_time], axis=1)
    boh = jax.nn.one_hot(buffer_type[:, 0], 2, dtype=f32)

    i_l2p = link_to_path.reshape(-1).astype(jnp.int32)
    p2l_f = path_to_link[:, :, 0].astype(jnp.int32)
    p2l_pos = path_to_link[:, :, 1].astype(jnp.int32)
    i_f9 = (p2l_f * 9 + p2l_pos).reshape(-1)
    i_trf = p2l_f.reshape(-1)
    i_q2l = queue_to_link[:, 0].astype(jnp.int32)

    tr16 = jnp.pad(flow_traffic, ((0, 0), (0, 15)))
    cap16 = jnp.pad(link_capacity, ((0, 0), (0, 15)))
    trg = _sc_gather(tr16, i_trf)[:, :1].reshape(_L, _K)
    capg = _sc_gather(cap16, i_l2p)[:, :1].reshape(_F, _P)

    path_state = _mlp_embed(path_in, pr['pe_W1'], pr['pe_b1'],
                            pr['pe_W2'], pr['pe_b2'], bf=2000)
    link_state, queue_state, T = _init_small(trg, link_capacity, boh, pr)

    S = jnp.tile(jnp.eye(_D, dtype=f32), (_K, 1))
    pss = None
    for _ in range(_ITER):
        xg = _sc_gather(T, i_l2p).reshape(_F, _P * 96)
        pss, path_state = _path_gru(xg, path_state, pr['gru_p_U'])
        psr = _sc_gather(pss.reshape(_F * 9, _D), i_f9).reshape(_L, _K * _D)
        queue_state, qw2 = _queue_gru(psr, queue_state, S, pr['gru_q_W'],
                                      pr['gru_q_U'], pr['gru_q_b'],
                                      pr['gru_l_W'], pr['gru_l_b'])
        qg2 = _sc_gather(qw2, i_q2l)
        link_state, T = _link_gru(qg2, link_state, queue_state, pr['gru_l_U'],
                                  pr['gru_p_W'][:_D], pr['gru_p_W'][_D:],
                                  pr['gru_p_b'])

    W1b = jsl.block_diag(*([pr['ro_W1']] * _P))
    b1b = jnp.tile(pr['ro_b1'], _P).reshape(1, -1)
    W2b = jsl.block_diag(*([pr['ro_W2']] * _P))
    b2b = jnp.tile(pr['ro_b2'], _P).reshape(1, -1)
    W3b = jsl.block_diag(*([pr['ro_W3']] * _P))
    b3b = jnp.tile(pr['ro_b3'], _P).reshape(1, -1)
    return _readout(pss, capg, flow_traffic, flow_packets,
                    W1b, b1b, W2b, b2b, W3b, b3b)


# speedup vs baseline: 5.2590x; 1.1664x over previous
import functools

import jax
import jax.numpy as jnp
import jax.scipy.linalg as jsl
from jax import lax
from jax.experimental import pallas as pl
from jax.experimental.pallas import tpu as pltpu
from jax.experimental.pallas import tpu_sc as plsc

_F = 50000
_L = 5000
_P = 8
_K = 80
_D = 32
_ITER = 8
_NW = 32


def _sc_gather(table, idx, fire=4):
    R = idx.shape[0]
    Dd = table.shape[1]
    assert R % 8 == 0
    span = -(-R // _NW)
    span = -(-span // 8) * 8
    min_count = R - (_NW - 1) * span
    assert min_count > 0
    ch = min(128, (min_count // 8) * 8)
    idx_pad = jnp.pad(idx, (0, _NW * span - R))
    mesh = plsc.VectorSubcoreMesh(core_axis_name="c", subcore_axis_name="s")

    @functools.partial(
        pl.kernel,
        mesh=mesh,
        out_type=jax.ShapeDtypeStruct((R, Dd), jnp.float32),
        compiler_params=pltpu.CompilerParams(use_tc_tiling_on_sc=False),
        scratch_types=(
            [pltpu.VMEM((span,), jnp.int32)]
            + [pltpu.VMEM((ch, Dd), jnp.float32) for _ in range(fire)]
            + [pltpu.SemaphoreType.DMA]
        ),
    )
    def gather_kernel(table_hbm, idx_hbm, out_hbm, idx_v, *rest):
        bufs, sem = rest[:fire], rest[fire]
        wid = lax.axis_index("s") * 2 + lax.axis_index("c")
        base = wid * span
        count = jnp.minimum(base + span, R) - base
        pltpu.sync_copy(idx_hbm.at[pl.ds(base, span)], idx_v)
        nfull = count // ch
        rem = count - nfull * ch
        nf = nfull // fire

        def gather_chunk(pos, buf):
            return pltpu.async_copy(
                table_hbm.at[idx_v.at[pl.ds(pos, ch)]], buf, sem)

        def store_chunk(pos, buf):
            pltpu.sync_copy(buf, out_hbm.at[pl.ds(base + pos, ch)])

        def body(j, carry):
            p0 = j * (fire * ch)
            descs = [gather_chunk(p0 + u * ch, bufs[u]) for u in range(fire)]
            for d in descs:
                d.wait()
            for u in range(fire):
                store_chunk(p0 + u * ch, bufs[u])
            return carry

        lax.fori_loop(0, nf, body, 0)
        done = nf * fire
        for u in range(fire - 1):
            @pl.when(done + u < nfull)
            def _(u=u):
                p = (done + u) * ch
                gather_chunk(p, bufs[0]).wait()
                store_chunk(p, bufs[0])

        @pl.when(rem > 0)
        def _():
            p = count - ch
            gather_chunk(p, bufs[0]).wait()
            store_chunk(p, bufs[0])

    return gather_kernel(table, idx_pad)


def _sc_gather_scalar(table, idx):
    R = idx.shape[0]
    V = table.shape[0]
    span = -(-R // _NW)
    span = -(-span // 16) * 16
    idx_pad = jnp.pad(idx, (0, _NW * span - R))
    mesh = plsc.VectorSubcoreMesh(core_axis_name="c", subcore_axis_name="s")

    @functools.partial(
        pl.kernel,
        mesh=mesh,
        out_type=jax.ShapeDtypeStruct((_NW * span,), jnp.float32),
        compiler_params=pltpu.CompilerParams(use_tc_tiling_on_sc=False,
                                             needs_layout_passes=False),
        scratch_types=[pltpu.VMEM((V,), jnp.float32),
                       pltpu.VMEM((span,), jnp.int32),
                       pltpu.VMEM((span,), jnp.float32)],
    )
    def gather_kernel(table_hbm, idx_hbm, out_hbm, tab_v, idx_v, out_v):
        wid = lax.axis_index("s") * 2 + lax.axis_index("c")
        base = wid * span
        pltpu.sync_copy(table_hbm, tab_v)
        pltpu.sync_copy(idx_hbm.at[pl.ds(base, span)], idx_v)

        def body(i, carry):
            sl = pl.ds(i * 16, 16)
            out_v[sl] = plsc.load_gather(tab_v, [idx_v[sl]])
            return carry

        lax.fori_loop(0, span // 16, body, 0)
        pltpu.sync_copy(out_v, out_hbm.at[pl.ds(base, span)])

    return gather_kernel(table, idx_pad)


def _full(shape):
    return pl.BlockSpec(shape, lambda i: tuple(0 for _ in shape))


def _mlp_embed(x, W1, b1, W2, b2, bf):
    n, din = x.shape
    d1, d2 = W1.shape[1], W2.shape[1]

    def body(x_ref, w1, bb1, w2, bb2, o_ref):
        h = jnp.maximum(
            jnp.dot(x_ref[...], w1[...],
                    preferred_element_type=jnp.float32) + bb1[...], 0.0)
        o_ref[...] = jnp.maximum(
            jnp.dot(h, w2[...],
                    preferred_element_type=jnp.float32) + bb2[...], 0.0)

    return pl.pallas_call(
        body,
        grid=(n // bf,),
        in_specs=[pl.BlockSpec((bf, din), lambda i: (i, 0)),
                  _full((din, d1)), _full((1, d1)),
                  _full((d1, d2)), _full((1, d2))],
        out_specs=pl.BlockSpec((bf, d2), lambda i: (i, 0)),
        out_shape=jax.ShapeDtypeStruct((n, d2), jnp.float32),
    )(x, W1, b1.reshape(1, -1), W2, b2.reshape(1, -1))


def _init_small(trg, cap, boh, pr):
    def body(trg_ref, cap_ref, boh_ref, lw1, lb1, lw2, lb2,
             qw1, qb1, qw2, qb2, wpq, wpl, bp, link_ref, queue_ref, t_ref):
        load = jnp.sum(trg_ref[...], axis=1, keepdims=True) / cap_ref[...]
        h = jnp.maximum(jnp.dot(load, lw1[...],
                                preferred_element_type=jnp.float32) + lb1[...], 0.0)
        ls = jnp.maximum(jnp.dot(h, lw2[...],
                                 preferred_element_type=jnp.float32) + lb2[...], 0.0)
        h = jnp.maximum(jnp.dot(boh_ref[...], qw1[...],
                                preferred_element_type=jnp.float32) + qb1[...], 0.0)
        qs = jnp.maximum(jnp.dot(h, qw2[...],
                                 preferred_element_type=jnp.float32) + qb2[...], 0.0)
        link_ref[...] = ls
        queue_ref[...] = qs
        t_ref[...] = (jnp.dot(qs, wpq[...], preferred_element_type=jnp.float32)
                      + jnp.dot(ls, wpl[...], preferred_element_type=jnp.float32)
                      + bp[...])

    return pl.pallas_call(
        body,
        grid=(1,),
        in_specs=[_full((_L, _K)), _full((_L, 1)), _full((_L, 2)),
                  _full((1, _D)), _full((1, _D)), _full((_D, _D)), _full((1, _D)),
                  _full((2, _D)), _full((1, _D)), _full((_D, _D)), _full((1, _D)),
                  _full((_D, 3 * _D)), _full((_D, 3 * _D)), _full((1, 3 * _D))],
        out_specs=(_full((_L, _D)), _full((_L, _D)), _full((_L, 3 * _D))),
        out_shape=(jax.ShapeDtypeStruct((_L, _D), jnp.float32),
                   jax.ShapeDtypeStruct((_L, _D), jnp.float32),
                   jax.ShapeDtypeStruct((_L, 3 * _D), jnp.float32)),
    )(trg, cap, boh,
      pr['le_W1'], pr['le_b1'].reshape(1, -1), pr['le_W2'], pr['le_b2'].reshape(1, -1),
      pr['qe_W1'], pr['qe_b1'].reshape(1, -1), pr['qe_W2'], pr['qe_b2'].reshape(1, -1),
      pr['gru_p_W'][:_D], pr['gru_p_W'][_D:], pr['gru_p_b'].reshape(1, -1))


def _gru_gates(xg, h, hg):
    z = jax.nn.sigmoid(xg[:, 0:_D] + hg[:, 0:_D])
    r = jax.nn.sigmoid(xg[:, _D:2 * _D] + hg[:, _D:2 * _D])
    c = jnp.tanh(xg[:, 2 * _D:3 * _D] + r * hg[:, 2 * _D:3 * _D])
    return z * h + (1.0 - z) * c


def _path_gru(xg, h0, U, bf=2000):
    def body(xg_ref, h_ref, u_ref, pss_ref, ho_ref):
        h = h_ref[...]
        u = u_ref[...]
        pss_ref[0] = h
        for t in range(_P):
            xgt = xg_ref[t]
            hg = jnp.dot(h, u, preferred_element_type=jnp.float32)
            h = _gru_gates(xgt, h, hg)
            pss_ref[t + 1] = h
        ho_ref[...] = h

    return pl.pallas_call(
        body,
        grid=(_F // bf,),
        in_specs=[pl.BlockSpec((_P, bf, 96), lambda i: (0, i, 0)),
                  pl.BlockSpec((bf, _D), lambda i: (i, 0)),
                  _full((_D, 3 * _D))],
        out_specs=(pl.BlockSpec((_P + 1, bf, _D), lambda i: (0, i, 0)),
                   pl.BlockSpec((bf, _D), lambda i: (i, 0))),
        out_shape=(jax.ShapeDtypeStruct((_P + 1, _F, _D), jnp.float32),
                   jax.ShapeDtypeStruct((_F, _D), jnp.float32)),
    )(xg, h0, U)


def _queue_gru(psr, q, Wq, Uq, bq, Wl, bl, bl_blk=1000):
    def body(psr_ref, q_ref, wq, uq, bbq, wl, bbl, qn_ref, qw2_ref):
        s = psr_ref[0]
        for k in range(1, _K // 4):
            s = s + psr_ref[k]
        psum = (s[:, 0:_D] + s[:, _D:2 * _D]
                + s[:, 2 * _D:3 * _D] + s[:, 3 * _D:4 * _D])
        xg = jnp.dot(psum, wq[...], preferred_element_type=jnp.float32) + bbq[...]
        qh = q_ref[...]
        hg = jnp.dot(qh, uq[...], preferred_element_type=jnp.float32)
        qn = _gru_gates(xg, qh, hg)
        qn_ref[...] = qn
        qw2_ref[...] = jnp.dot(qn, wl[...],
                               preferred_element_type=jnp.float32) + bbl[...]

    return pl.pallas_call(
        body,
        grid=(_L // bl_blk,),
        in_specs=[pl.BlockSpec((_K // 4, bl_blk, 4 * _D), lambda i: (0, i, 0)),
                  pl.BlockSpec((bl_blk, _D), lambda i: (i, 0)),
                  _full((_D, 3 * _D)), _full((_D, 3 * _D)), _full((1, 3 * _D)),
                  _full((_D, 3 * _D)), _full((1, 3 * _D))],
        out_specs=(pl.BlockSpec((bl_blk, _D), lambda i: (i, 0)),
                   pl.BlockSpec((bl_blk, 3 * _D), lambda i: (i, 0))),
        out_shape=(jax.ShapeDtypeStruct((_L, _D), jnp.float32),
                   jax.ShapeDtypeStruct((_L, 3 * _D), jnp.float32)),
    )(psr, q, Wq, Uq, bq.reshape(1, -1), Wl, bl.reshape(1, -1))


def _link_gru(qg2, lnk, qn, Ul, WpQ, WpL, bp):
    def body(qg_ref, l_ref, qn_ref, ul, wpq, wpl, bbp, lo_ref, t_ref):
        hl = l_ref[...]
        hg = jnp.dot(hl, ul[...], preferred_element_type=jnp.float32)
        ln = _gru_gates(qg_ref[...], hl, hg)
        lo_ref[...] = ln
        t_ref[...] = (jnp.dot(qn_ref[...], wpq[...],
                              preferred_element_type=jnp.float32)
                      + jnp.dot(ln, wpl[...], preferred_element_type=jnp.float32)
                      + bbp[...])

    return pl.pallas_call(
        body,
        grid=(1,),
        in_specs=[_full((_L, 3 * _D)), _full((_L, _D)), _full((_L, _D)),
                  _full((_D, 3 * _D)), _full((_D, 3 * _D)), _full((_D, 3 * _D)),
                  _full((1, 3 * _D))],
        out_specs=(_full((_L, _D)), _full((_L, 3 * _D))),
        out_shape=(jax.ShapeDtypeStruct((_L, _D), jnp.float32),
                   jax.ShapeDtypeStruct((_L, 3 * _D), jnp.float32)),
    )(qg2, lnk, qn, Ul, WpQ, WpL, bp.reshape(1, -1))


def _readout(pss, capg, tr, pk, W1b, b1b, W2b, b2b, W3b, b3b, bf=2000):
    def body(pss_ref, cg_ref, tr_ref, pk_ref, w1, bb1, w2, bb2, w3, bb3, o_ref):
        x = jnp.concatenate([pss_ref[t] for t in range(1, _P + 1)], axis=1)
        h1 = jnp.maximum(jnp.dot(x, w1[...],
                                 preferred_element_type=jnp.float32) + bb1[...], 0.0)
        h2 = jnp.maximum(jnp.dot(h1, w2[...],
                                 preferred_element_type=jnp.float32) + bb2[...], 0.0)
        occ = jnp.dot(h2, w3[...], preferred_element_type=jnp.float32) + bb3[...]
        cg = cg_ref[...] * 1e9
        qd = jnp.sum(occ / cg, axis=1, keepdims=True)
        inv = jnp.sum(1.0 / cg, axis=1, keepdims=True)
        o_ref[...] = qd + (tr_ref[...] / pk_ref[...]) * inv

    return pl.pallas_call(
        body,
        grid=(_F // bf,),
        in_specs=[pl.BlockSpec((_P + 1, bf, _D), lambda i: (0, i, 0)),
                  pl.BlockSpec((bf, _P), lambda i: (i, 0)),
                  pl.BlockSpec((bf, 1), lambda i: (i, 0)),
                  pl.BlockSpec((bf, 1), lambda i: (i, 0)),
                  _full((8 * _D, 128)), _full((1, 128)),
                  _full((128, 128)), _full((1, 128)),
                  _full((128, _P)), _full((1, _P))],
        out_specs=pl.BlockSpec((bf, 1), lambda i: (i, 0)),
        out_shape=jax.ShapeDtypeStruct((_F, 1), jnp.float32),
    )(pss, capg, tr, pk, W1b, b1b, W2b, b2b, W3b, b3b)


def kernel(flow_traffic, flow_packets, flow_time_dist, flow_lambda,
           flow_ON_bits_rate, flow_ON_time, flow_OFF_time, link_capacity,
           buffer_type, link_to_path, path_to_link, queue_to_link, params):
    pr = params
    f32 = jnp.float32

    dist_oh = jax.nn.one_hot(flow_time_dist[:, 0], 3, dtype=f32)
    path_in = jnp.concatenate(
        [flow_traffic, flow_packets, dist_oh, flow_lambda,
         flow_ON_bits_rate, flow_ON_time, flow_OFF_time], axis=1)
    boh = jax.nn.one_hot(buffer_type[:, 0], 2, dtype=f32)

    l2p = link_to_path.astype(jnp.int32)
    i_l2p_t = l2p.T.reshape(-1)
    i_l2p = l2p.reshape(-1)
    p2l_f = path_to_link[:, :, 0].astype(jnp.int32)
    p2l_pos = path_to_link[:, :, 1].astype(jnp.int32)
    i_f9_t = ((p2l_pos * _F + p2l_f)
              .reshape(_L, _K // 4, 4).transpose(1, 0, 2).reshape(-1))
    i_trf = p2l_f.reshape(-1)
    i_q2l = queue_to_link[:, 0].astype(jnp.int32)

    trg = _sc_gather_scalar(flow_traffic[:, 0], i_trf)[:_L * _K]
    trg = trg.reshape(_L, _K)
    capg = _sc_gather_scalar(link_capacity[:, 0], i_l2p)[:_F * _P]
    capg = capg.reshape(_F, _P)

    path_state = _mlp_embed(path_in, pr['pe_W1'], pr['pe_b1'],
                            pr['pe_W2'], pr['pe_b2'], bf=2000)
    link_state, queue_state, T = _init_small(trg, link_capacity, boh, pr)

    pss = None
    for _ in range(_ITER):
        xg = _sc_gather(T, i_l2p_t).reshape(_P, _F, 96)
        pss, path_state = _path_gru(xg, path_state, pr['gru_p_U'])
        psr = _sc_gather(pss.reshape((_P + 1) * _F, _D), i_f9_t)
        psr = psr.reshape(_K // 4, _L, 4 * _D)
        queue_state, qw2 = _queue_gru(psr, queue_state, pr['gru_q_W'],
                                      pr['gru_q_U'], pr['gru_q_b'],
                                      pr['gru_l_W'], pr['gru_l_b'])
        qg2 = _sc_gather(qw2, i_q2l)
        link_state, T = _link_gru(qg2, link_state, queue_state, pr['gru_l_U'],
                                  pr['gru_p_W'][:_D], pr['gru_p_W'][_D:],
                                  pr['gru_p_b'])

    W1b = jsl.block_diag(*([pr['ro_W1']] * _P))
    b1b = jnp.tile(pr['ro_b1'], _P).reshape(1, -1)
    W2b = jsl.block_diag(*([pr['ro_W2']] * _P))
    b2b = jnp.tile(pr['ro_b2'], _P).reshape(1, -1)
    W3b = jsl.block_diag(*([pr['ro_W3']] * _P))
    b3b = jnp.tile(pr['ro_b3'], _P).reshape(1, -1)
    return _readout(pss, capg, flow_traffic, flow_packets,
                    W1b, b1b, W2b, b2b, W3b, b3b)
